# baseline (speedup 1.0000x reference)
import functools

import jax
import jax.numpy as jnp
from jax import lax
from jax.experimental import pallas as pl
from jax.experimental.pallas import tpu as pltpu
from jax.experimental.pallas import tpu_sc as plsc

_N = 10000
_E = 320000
_B = 64
_OUT = 6630
_NC, _NS, _LANES = 2, 16, 16
_NW = _NC * _NS
_EPT = _E // _NW
_CH = 160
_NCH = (_E // _NS) // _CH
_NPAD = 10240
_RPT = _NPAD // _NS
_RW = 128
_ROWBLK = 1000
_GRID = _N // _ROWBLK
_CBLK = 512
_CGRID = (_OUT + _CBLK - 1) // _CBLK


def _sc_mesh():
    return plsc.VectorSubcoreMesh(core_axis_name="c", subcore_axis_name="s")


def _sc_deg_body(dstv, wv, out, dst_b, w_b, deg_b):
    c = lax.axis_index("c")
    s = lax.axis_index("s")
    wid = c * _NS + s

    def zero_row(i, t):
        deg_b[pl.ds(i * _LANES, _LANES)] = jnp.zeros((_LANES,), jnp.float32)
        return t

    lax.fori_loop(0, _N // _LANES, zero_row, 0)

    e0 = pl.multiple_of(wid * _EPT, 8)
    pltpu.sync_copy(dstv.at[pl.ds(e0, _EPT)], dst_b)
    pltpu.sync_copy(wv.at[pl.ds(e0, _EPT)], w_b)

    def scat(i, t):
        sl = pl.ds(i * _LANES, _LANES)
        plsc.addupdate_scatter(deg_b, [dst_b[sl]], w_b[sl])
        return t

    lax.fori_loop(0, _EPT // _LANES, scat, 0)
    pltpu.sync_copy(deg_b, out.at[pl.ds(pl.multiple_of(wid * _N, 8), _N)])


def _sc_deg(dst, w):
    fn = pl.kernel(
        _sc_deg_body,
        out_type=jax.ShapeDtypeStruct((_NW * _N,), jnp.float32),
        mesh=_sc_mesh(),
        compiler_params=pltpu.CompilerParams(needs_layout_passes=False),
        scratch_types=[
            pltpu.VMEM((_EPT,), jnp.int32),
            pltpu.VMEM((_EPT,), jnp.float32),
            pltpu.VMEM((_N,), jnp.float32),
        ],
    )
    return fn(dst, w)


def _scale_rows(rows, ebuf):
    def grp(g, t):
        w16 = plsc.bitcast(ebuf[2, pl.ds(g * _LANES, _LANES)], jnp.float32)
        for r in range(_LANES):
            wb = lax.gather(
                w16, jnp.full((_LANES, 1), r, jnp.int32),
                lax.GatherDimensionNumbers(offset_dims=(),
                                           collapsed_slice_dims=(0,),
                                           start_index_map=(0,)),
                slice_sizes=(1,),
                mode=lax.GatherScatterMode.PROMISE_IN_BOUNDS)
            row = g * _LANES + r
            for j in range(4):
                sl = pl.ds(j * _LANES, _LANES)
                rows[row, sl] = rows[row, sl] * wb
        return t

    lax.fori_loop(0, _CH // _LANES, grp, 0)


_NBUF = 4
_ZR = 160


def _sc_agg_body(ytab, edges, out, eb0, eb1, eb2, eb3, rb0, rb1, rb2, rb3,
                 acc, isem, gs0, gs1, gs2, gs3, ss0, ss1, ss2, ss3):
    c = lax.axis_index("c")
    s = lax.axis_index("s")
    ebuf = (eb0, eb1, eb2, eb3)
    rows = (rb0, rb1, rb2, rb3)
    gsem = (gs0, gs1, gs2, gs3)
    ssem = (ss0, ss1, ss2, ss3)
    coff = c * _N

    def zero_row(i, t):
        for j in range(4):
            rb0[i, pl.ds(j * _LANES, _LANES)] = jnp.zeros((_LANES,),
                                                          jnp.float32)
        return t

    lax.fori_loop(0, _ZR, zero_row, 0)
    for k in range(_RPT // _ZR):
        pltpu.sync_copy(rb0, acc.at[pl.ds(s * _RPT + k * _ZR, _ZR)])
    plsc.subcore_barrier()

    g0 = s * _NCH

    def idx_load(k, b):
        pltpu.async_copy(edges.at[g0 + k], ebuf[b], isem)

    def idx_wait(k, b):
        pltpu.make_async_copy(edges.at[g0 + k], ebuf[b], isem).wait()
        def fix(g, t):
            sl = pl.ds(g * _LANES, _LANES)
            ebuf[b][0, sl] = ebuf[b][0, sl] + coff
            return t
        lax.fori_loop(0, _CH // _LANES, fix, 0)

    def gather(k, b):
        pltpu.async_copy(ytab.at[ebuf[b].at[0]], rows[b], gsem[b])

    def gather_wait(k, b):
        pltpu.make_async_copy(ytab.at[ebuf[b].at[0]], rows[b], gsem[b]).wait()

    def scatter(k, b):
        pltpu.async_copy(rows[b], acc.at[ebuf[b].at[1]], ssem[b], add=True)

    def scatter_wait(k, b):
        pltpu.make_async_copy(rows[b], acc.at[ebuf[b].at[1]], ssem[b]).wait()

    def step(k, b, bn, drain):
        if drain:
            scatter_wait(k - 2, bn)
        idx_load(k + 2, bn)
        gather_wait(k, b)
        _scale_rows(rows[b], ebuf[b])
        scatter(k, b)
        idx_wait(k + 2, bn)
        gather(k + 2, bn)

    pltpu.sync_copy(edges.at[g0], ebuf[0])
    idx_wait_sync0 = None
    def fix0(b):
        def fix(g, t):
            sl = pl.ds(g * _LANES, _LANES)
            ebuf[b][0, sl] = ebuf[b][0, sl] + coff
            return t
        lax.fori_loop(0, _CH // _LANES, fix, 0)
    fix0(0)
    gather(0, 0)
    pltpu.sync_copy(edges.at[g0 + 1], ebuf[1])
    fix0(1)
    gather(1, 1)
    step(0, 0, 2, drain=False)
    step(1, 1, 3, drain=False)

    def quad(m, t):
        k = 2 + 4 * m
        for b in range(_NBUF):
            bb = (2 + b) % _NBUF
            step(k + b, bb, b, drain=True)
        return t

    lax.fori_loop(0, (_NCH - 2) // 4, quad, 0)
    for kk in range(_NCH - 3, _NCH):
        b = kk % _NBUF
        scatter_wait(kk - 2, (kk + 2) % _NBUF)
        if kk + 2 < _NCH:
            idx_load(kk + 2, (kk + 2) % _NBUF)
        gather_wait(kk, b)
        _scale_rows(rows[b], ebuf[b])
        scatter(kk, b)
        if kk + 2 < _NCH:
            idx_wait(kk + 2, (kk + 2) % _NBUF)
            gather(kk + 2, (kk + 2) % _NBUF)
    scatter_wait(_NCH - 2, (_NCH - 2) % _NBUF)
    scatter_wait(_NCH - 1, (_NCH - 1) % _NBUF)

    plsc.subcore_barrier()
    r0 = s * _RPT
    pltpu.sync_copy(acc.at[pl.ds(r0, _RPT)], out.at[pl.ds(c * _NPAD + r0,
                                                          _RPT)])


def _sc_agg(ytab2, edges):
    fn = pl.kernel(
        _sc_agg_body,
        out_type=jax.ShapeDtypeStruct((_NC * _NPAD, 64), jnp.float32),
        mesh=_sc_mesh(),
        compiler_params=pltpu.CompilerParams(needs_layout_passes=False,
                                             use_tc_tiling_on_sc=False),
        scratch_types=[
            pltpu.VMEM((3, _CH), jnp.int32),
            pltpu.VMEM((3, _CH), jnp.int32),
            pltpu.VMEM((3, _CH), jnp.int32),
            pltpu.VMEM((3, _CH), jnp.int32),
            pltpu.VMEM((_CH, 64), jnp.float32),
            pltpu.VMEM((_CH, 64), jnp.float32),
            pltpu.VMEM((_CH, 64), jnp.float32),
            pltpu.VMEM((_CH, 64), jnp.float32),
            pltpu.VMEM_SHARED((_NPAD, 64), jnp.float32),
        ] + [pltpu.SemaphoreType.DMA] * 9,
    )
    return fn(ytab2, edges)


def _tc_prep_body(degp, dinv_o, sdeg_o):
    deg = jnp.sum(degp[...], axis=0, keepdims=True) + 1.0
    dinv_o[...] = lax.rsqrt(deg)
    sdeg_o[...] = jnp.sqrt(deg)


def _tc_prep(degp):
    return pl.pallas_call(
        _tc_prep_body,
        out_shape=[jax.ShapeDtypeStruct((1, _N), jnp.float32)] * 2,
    )(degp)


def _tc_scale_body(x, dinv, o):
    o[0] = x[:, :64] * dinv[...]
    o[1] = x[:, 64:] * dinv[...]


def _tc_scale(x, dinv_col):
    return pl.pallas_call(
        _tc_scale_body,
        grid=(_GRID,),
        in_specs=[pl.BlockSpec((_ROWBLK, 128), lambda i: (i, 0)),
                  pl.BlockSpec((_ROWBLK, 1), lambda i: (i, 0))],
        out_specs=pl.BlockSpec((_NC, _ROWBLK, 64), lambda i: (0, i, 0)),
        out_shape=jax.ShapeDtypeStruct((_NC, _N, 64), jnp.float32),
    )(x, dinv_col)


def _layer_body(t_in, t_out, *refs):
    zps = refs[:t_in]
    ys = refs[t_in:2 * t_in]
    dinv, wref, bref = refs[2 * t_in:2 * t_in + 3]
    outs = refs[2 * t_in + 3:]
    w = wref[...]
    d = dinv[...]
    acc = None
    for t in range(t_in):
        for half in range(2):
            agg = (zps[t][half] + ys[t][half]) * d
            k0 = t * 128 + half * 64
            p = jnp.dot(agg, w[k0:k0 + 64, :],
                        preferred_element_type=jnp.float32)
            acc = p if acc is None else acc + p
    h = jnp.maximum(acc + bref[...], 0.0)
    for t in range(t_out):
        outs[t][0] = h[:, t * 128:t * 128 + 64] * d
        outs[t][1] = h[:, t * 128 + 64:(t + 1) * 128] * d


def _tc_layer(zps, ys, dinv_col, W, b2d):
    t_in = len(ys)
    f_in, f_out = W.shape
    t_out = f_out // 128
    body = functools.partial(_layer_body, t_in, t_out)
    in_specs = (
        [pl.BlockSpec((_NC, _ROWBLK, 64), lambda i: (0, i, 0))] * t_in
        + [pl.BlockSpec((_NC, _ROWBLK, 64), lambda i: (0, i, 0))] * t_in
        + [pl.BlockSpec((_ROWBLK, 1), lambda i: (i, 0)),
           pl.BlockSpec((f_in, f_out), lambda i: (0, 0)),
           pl.BlockSpec((1, f_out), lambda i: (0, 0))]
    )
    out = pl.pallas_call(
        body,
        grid=(_GRID,),
        in_specs=in_specs,
        out_specs=[pl.BlockSpec((_NC, _ROWBLK, 64),
                                lambda i: (0, i, 0))] * t_out,
        out_shape=[jax.ShapeDtypeStruct((_NC, _N, 64), jnp.float32)] * t_out,
    )(*zps, *ys, dinv_col, W, b2d)
    return list(out)


def _pool_body(*refs):
    nt = 15
    ys = refs[:nt]
    sdeg, batch = refs[nt], refs[nt + 1]
    out = refs[nt + 2]
    cnt = refs[nt + 3]
    i = pl.program_id(0)
    iota = lax.broadcasted_iota(jnp.int32, (_ROWBLK, _B), 1)
    mask = (batch[...] == iota).astype(jnp.float32)
    m2 = mask * sdeg[...]

    @pl.when(i == 0)
    def _():
        out[...] = jnp.zeros((_B, nt * 128), jnp.float32)
        cnt[...] = jnp.zeros((_B, 128), jnp.float32)

    cnt[...] += lax.dot_general(mask, jnp.ones((_ROWBLK, 128), jnp.float32),
                                (((0,), (0,)), ((), ())),
                                preferred_element_type=jnp.float32)
    for t in range(nt):
        for half in range(2):
            c = lax.dot_general(m2, ys[t][half], (((0,), (0,)), ((), ())),
                                preferred_element_type=jnp.float32)
            c0 = t * 128 + half * 64
            out[:, c0:c0 + 64] += c

    @pl.when(i == _GRID - 1)
    def _():
        out[...] = out[...] / jnp.clip(cnt[:, 0:1], 1.0, None)


def _tc_pool(tiles, sdeg_col, batch_col):
    nt = len(tiles)
    in_specs = (
        [pl.BlockSpec((_NC, _ROWBLK, 64), lambda i: (0, i, 0))] * nt
        + [pl.BlockSpec((_ROWBLK, 1), lambda i: (i, 0))] * 2
    )
    return pl.pallas_call(
        _pool_body,
        grid=(_GRID,),
        in_specs=in_specs,
        out_specs=pl.BlockSpec((_B, nt * 128), lambda i: (0, 0)),
        out_shape=jax.ShapeDtypeStruct((_B, nt * 128), jnp.float32),
        scratch_shapes=[pltpu.VMEM((_B, 128), jnp.float32)],
    )(*tiles, sdeg_col, batch_col)


def _final_body(pooled, wl, bl, out):
    out[...] = jnp.dot(pooled[...], wl[...],
                       preferred_element_type=jnp.float32) + bl[...]


def _tc_final(pooled, Wl, bl2d):
    return pl.pallas_call(
        _final_body,
        grid=(_CGRID,),
        in_specs=[pl.BlockSpec((_B, 1920), lambda j: (0, 0)),
                  pl.BlockSpec((1920, _CBLK), lambda j: (0, j)),
                  pl.BlockSpec((1, _CBLK), lambda j: (0, j))],
        out_specs=pl.BlockSpec((_B, _CBLK), lambda j: (0, j)),
        out_shape=jax.ShapeDtypeStruct((_B, _OUT), jnp.float32),
    )(pooled, Wl, bl2d)


def kernel(x, edge_index, edge_attr, batch,
           W1, b1, W2, b2, W3, b3, W4, b4, Wl, bl):
    src = edge_index[0]
    dst = edge_index[1]
    edges = jnp.stack([src.reshape(_E // _CH, _CH),
                       dst.reshape(_E // _CH, _CH),
                       lax.bitcast_convert_type(edge_attr, jnp.int32).reshape(_E // _CH, _CH)],
                      axis=1)
    degp = _sc_deg(dst, edge_attr).reshape(_NW, _N)
    dinv2d, sdeg2d = _tc_prep(degp)
    dinv_col = dinv2d.reshape(_N, 1)
    sdeg_col = sdeg2d.reshape(_N, 1)
    y1 = _tc_scale(x, dinv_col)
    cur = [y1]
    pool_tiles = []
    for (W, b) in ((W1, b1), (W2, b2), (W3, b3), (W4, b4)):
        zps = [_sc_agg(t.reshape(_NC * _N, 64), edges).reshape(_NC, _NPAD, 64)
               for t in cur]
        cur = _tc_layer(zps, cur, dinv_col, W, b.reshape(1, -1))
        pool_tiles.extend(cur)
    pooled = _tc_pool(pool_tiles, sdeg_col, batch.reshape(_N, 1))
    return _tc_final(pooled, Wl, bl.reshape(1, -1))


# speedup vs baseline: 2.5540x; 2.5540x over previous
import functools

import jax
import jax.numpy as jnp
from jax import lax
from jax.experimental import pallas as pl
from jax.experimental.pallas import tpu as pltpu
from jax.experimental.pallas import tpu_sc as plsc

_N = 10000
_E = 320000
_B = 64
_OUT = 6630
_NC, _NS, _LANES = 2, 16, 16
_NW = _NC * _NS
_EPT = _E // _NW
_CH = 80
_NCH = _EPT // _CH
_NPAD = 10240
_RPT = _NPAD // _NS
_RW = 128
_ROWBLK = 1000
_GRID = _N // _ROWBLK
_CBLK = 512
_CGRID = (_OUT + _CBLK - 1) // _CBLK


def _sc_mesh():
    return plsc.VectorSubcoreMesh(core_axis_name="c", subcore_axis_name="s")


def _sc_deg_body(dstv, wv, out, dst_b, w_b, deg_b):
    c = lax.axis_index("c")
    s = lax.axis_index("s")
    wid = c * _NS + s

    def zero_row(i, t):
        deg_b[pl.ds(i * _LANES, _LANES)] = jnp.zeros((_LANES,), jnp.float32)
        return t

    lax.fori_loop(0, _N // _LANES, zero_row, 0)

    e0 = pl.multiple_of(wid * _EPT, 8)
    pltpu.sync_copy(dstv.at[pl.ds(e0, _EPT)], dst_b)
    pltpu.sync_copy(wv.at[pl.ds(e0, _EPT)], w_b)

    def scat(i, t):
        sl = pl.ds(i * _LANES, _LANES)
        plsc.addupdate_scatter(deg_b, [dst_b[sl]], w_b[sl])
        return t

    lax.fori_loop(0, _EPT // _LANES, scat, 0)
    pltpu.sync_copy(deg_b, out.at[pl.ds(pl.multiple_of(wid * _N, 8), _N)])


def _sc_deg(dst, w):
    fn = pl.kernel(
        _sc_deg_body,
        out_type=jax.ShapeDtypeStruct((_NW * _N,), jnp.float32),
        mesh=_sc_mesh(),
        compiler_params=pltpu.CompilerParams(needs_layout_passes=False),
        scratch_types=[
            pltpu.VMEM((_EPT,), jnp.int32),
            pltpu.VMEM((_EPT,), jnp.float32),
            pltpu.VMEM((_N,), jnp.float32),
        ],
    )
    return fn(dst, w)


def _scale_rows(rows, ebuf):
    def grp(g, t):
        w16 = plsc.bitcast(ebuf[2, pl.ds(g * _LANES, _LANES)], jnp.float32)
        for r in range(_LANES):
            wb = lax.gather(
                w16, jnp.full((_LANES, 1), r, jnp.int32),
                lax.GatherDimensionNumbers(offset_dims=(),
                                           collapsed_slice_dims=(0,),
                                           start_index_map=(0,)),
                slice_sizes=(1,),
                mode=lax.GatherScatterMode.PROMISE_IN_BOUNDS)
            row = g * _LANES + r
            for j in range(8):
                sl = pl.ds(j * _LANES, _LANES)
                rows[row, sl] = rows[row, sl] * wb
        return t

    lax.fori_loop(0, _CH // _LANES, grp, 0)


_NBUF = 4
_ZR = 80


def _sc_agg_body(ytab, edges, out, eb0, eb1, eb2, eb3, rb0, rb1, rb2, rb3,
                 acc, isem, gs0, gs1, gs2, gs3, ss0, ss1, ss2, ss3):
    c = lax.axis_index("c")
    s = lax.axis_index("s")
    ebuf = (eb0, eb1, eb2, eb3)
    rows = (rb0, rb1, rb2, rb3)
    gsem = (gs0, gs1, gs2, gs3)
    ssem = (ss0, ss1, ss2, ss3)

    def zero_row(i, t):
        for j in range(8):
            rb0[i, pl.ds(j * _LANES, _LANES)] = jnp.zeros((_LANES,),
                                                          jnp.float32)
        return t

    lax.fori_loop(0, _ZR, zero_row, 0)
    for k in range(_RPT // _ZR):
        pltpu.sync_copy(rb0, acc.at[pl.ds(s * _RPT + k * _ZR, _ZR)])
    plsc.subcore_barrier()

    wid = c * _NS + s
    g0 = wid * _NCH

    def idx_load(k, b):
        pltpu.async_copy(edges.at[g0 + k], ebuf[b], isem)

    def idx_wait(k, b):
        pltpu.make_async_copy(edges.at[g0 + k], ebuf[b], isem).wait()

    def gather(k, b):
        pltpu.async_copy(ytab.at[ebuf[b].at[0]], rows[b], gsem[b])

    def gather_wait(k, b):
        pltpu.make_async_copy(ytab.at[ebuf[b].at[0]], rows[b], gsem[b]).wait()

    def scatter(k, b):
        pltpu.async_copy(rows[b], acc.at[ebuf[b].at[1]], ssem[b], add=True)

    def scatter_wait(k, b):
        pltpu.make_async_copy(rows[b], acc.at[ebuf[b].at[1]], ssem[b]).wait()

    def step(k, b, bn, drain):
        if drain:
            scatter_wait(k - 2, bn)
        idx_load(k + 2, bn)
        gather_wait(k, b)
        _scale_rows(rows[b], ebuf[b])
        scatter(k, b)
        idx_wait(k + 2, bn)
        gather(k + 2, bn)

    pltpu.sync_copy(edges.at[g0], ebuf[0])
    gather(0, 0)
    pltpu.sync_copy(edges.at[g0 + 1], ebuf[1])
    gather(1, 1)
    step(0, 0, 2, drain=False)
    step(1, 1, 3, drain=False)

    def quad(m, t):
        k = 2 + 4 * m
        for b in range(_NBUF):
            bb = (2 + b) % _NBUF
            step(k + b, bb, b, drain=True)
        return t

    lax.fori_loop(0, (_NCH - 2) // 4, quad, 0)
    for kk in range(_NCH - 3, _NCH):
        b = kk % _NBUF
        scatter_wait(kk - 2, (kk + 2) % _NBUF)
        if kk + 2 < _NCH:
            idx_load(kk + 2, (kk + 2) % _NBUF)
        gather_wait(kk, b)
        _scale_rows(rows[b], ebuf[b])
        scatter(kk, b)
        if kk + 2 < _NCH:
            idx_wait(kk + 2, (kk + 2) % _NBUF)
            gather(kk + 2, (kk + 2) % _NBUF)
    scatter_wait(_NCH - 2, (_NCH - 2) % _NBUF)
    scatter_wait(_NCH - 1, (_NCH - 1) % _NBUF)

    plsc.subcore_barrier()
    r0 = s * _RPT
    pltpu.sync_copy(acc.at[pl.ds(r0, _RPT)], out.at[pl.ds(c * _NPAD + r0,
                                                          _RPT)])


def _sc_agg(ytab, edges):
    fn = pl.kernel(
        _sc_agg_body,
        out_type=jax.ShapeDtypeStruct((_NC * _NPAD, 128), jnp.float32),
        mesh=_sc_mesh(),
        compiler_params=pltpu.CompilerParams(needs_layout_passes=False),
        scratch_types=[
            pltpu.VMEM((3, _CH), jnp.int32),
            pltpu.VMEM((3, _CH), jnp.int32),
            pltpu.VMEM((3, _CH), jnp.int32),
            pltpu.VMEM((3, _CH), jnp.int32),
            pltpu.VMEM((_CH, 128), jnp.float32),
            pltpu.VMEM((_CH, 128), jnp.float32),
            pltpu.VMEM((_CH, 128), jnp.float32),
            pltpu.VMEM((_CH, 128), jnp.float32),
            pltpu.VMEM_SHARED((_NPAD, 128), jnp.float32),
        ] + [pltpu.SemaphoreType.DMA] * 9,
    )
    return fn(ytab, edges)


def _tc_prep_body(degp, dinv_o, sdeg_o):
    deg = jnp.sum(degp[...], axis=0, keepdims=True) + 1.0
    dinv_o[...] = lax.rsqrt(deg)
    sdeg_o[...] = jnp.sqrt(deg)


def _tc_prep(degp):
    return pl.pallas_call(
        _tc_prep_body,
        out_shape=[jax.ShapeDtypeStruct((1, _N), jnp.float32)] * 2,
    )(degp)


def _tc_scale_body(x, dinv, o):
    o[...] = x[...] * dinv[...]


def _tc_scale(x, dinv_col):
    return pl.pallas_call(
        _tc_scale_body,
        grid=(_GRID,),
        in_specs=[pl.BlockSpec((_ROWBLK, 128), lambda i: (i, 0)),
                  pl.BlockSpec((_ROWBLK, 1), lambda i: (i, 0))],
        out_specs=pl.BlockSpec((_ROWBLK, 128), lambda i: (i, 0)),
        out_shape=jax.ShapeDtypeStruct((_N, 128), jnp.float32),
    )(x, dinv_col)


def _layer_body(t_in, t_out, *refs):
    zps = refs[:t_in]
    ys = refs[t_in:2 * t_in]
    dinv, wref, bref = refs[2 * t_in:2 * t_in + 3]
    outs = refs[2 * t_in + 3:]
    w = wref[...]
    d = dinv[...]
    acc = None
    for t in range(t_in):
        agg = (zps[t][0] + zps[t][1] + ys[t][...]) * d
        p = jnp.dot(agg, w[t * 128:(t + 1) * 128, :],
                    preferred_element_type=jnp.float32)
        acc = p if acc is None else acc + p
    h = jnp.maximum(acc + bref[...], 0.0)
    for t in range(t_out):
        outs[t][...] = h[:, t * 128:(t + 1) * 128] * d


def _tc_layer(zps, ys, dinv_col, W, b2d):
    t_in = len(ys)
    f_in, f_out = W.shape
    t_out = f_out // 128
    body = functools.partial(_layer_body, t_in, t_out)
    in_specs = (
        [pl.BlockSpec((_NC, _ROWBLK, 128), lambda i: (0, i, 0))] * t_in
        + [pl.BlockSpec((_ROWBLK, 128), lambda i: (i, 0))] * t_in
        + [pl.BlockSpec((_ROWBLK, 1), lambda i: (i, 0)),
           pl.BlockSpec((f_in, f_out), lambda i: (0, 0)),
           pl.BlockSpec((1, f_out), lambda i: (0, 0))]
    )
    out = pl.pallas_call(
        body,
        grid=(_GRID,),
        in_specs=in_specs,
        out_specs=[pl.BlockSpec((_ROWBLK, 128), lambda i: (i, 0))] * t_out,
        out_shape=[jax.ShapeDtypeStruct((_N, 128), jnp.float32)] * t_out,
    )(*zps, *ys, dinv_col, W, b2d)
    return list(out)


def _pool_body(*refs):
    nt = 15
    ys = refs[:nt]
    sdeg, batch = refs[nt], refs[nt + 1]
    out = refs[nt + 2]
    cnt = refs[nt + 3]
    i = pl.program_id(0)
    iota = lax.broadcasted_iota(jnp.int32, (_ROWBLK, _B), 1)
    mask = (batch[...] == iota).astype(jnp.float32)
    m2 = mask * sdeg[...]

    @pl.when(i == 0)
    def _():
        out[...] = jnp.zeros((_B, nt * 128), jnp.float32)
        cnt[...] = jnp.zeros((_B, 128), jnp.float32)

    cnt[...] += lax.dot_general(mask, jnp.ones((_ROWBLK, 128), jnp.float32),
                                (((0,), (0,)), ((), ())),
                                preferred_element_type=jnp.float32)
    for t in range(nt):
        c = lax.dot_general(m2, ys[t][...], (((0,), (0,)), ((), ())),
                            preferred_element_type=jnp.float32)
        out[:, t * 128:(t + 1) * 128] += c

    @pl.when(i == _GRID - 1)
    def _():
        out[...] = out[...] / jnp.clip(cnt[:, 0:1], 1.0, None)


def _tc_pool(tiles, sdeg_col, batch_col):
    nt = len(tiles)
    in_specs = (
        [pl.BlockSpec((_ROWBLK, 128), lambda i: (i, 0))] * nt
        + [pl.BlockSpec((_ROWBLK, 1), lambda i: (i, 0))] * 2
    )
    return pl.pallas_call(
        _pool_body,
        grid=(_GRID,),
        in_specs=in_specs,
        out_specs=pl.BlockSpec((_B, nt * 128), lambda i: (0, 0)),
        out_shape=jax.ShapeDtypeStruct((_B, nt * 128), jnp.float32),
        scratch_shapes=[pltpu.VMEM((_B, 128), jnp.float32)],
    )(*tiles, sdeg_col, batch_col)


def _final_body(pooled, wl, bl, out):
    out[...] = jnp.dot(pooled[...], wl[...],
                       preferred_element_type=jnp.float32) + bl[...]


def _tc_final(pooled, Wl, bl2d):
    return pl.pallas_call(
        _final_body,
        grid=(_CGRID,),
        in_specs=[pl.BlockSpec((_B, 1920), lambda j: (0, 0)),
                  pl.BlockSpec((1920, _CBLK), lambda j: (0, j)),
                  pl.BlockSpec((1, _CBLK), lambda j: (0, j))],
        out_specs=pl.BlockSpec((_B, _CBLK), lambda j: (0, j)),
        out_shape=jax.ShapeDtypeStruct((_B, _OUT), jnp.float32),
    )(pooled, Wl, bl2d)


def kernel(x, edge_index, edge_attr, batch,
           W1, b1, W2, b2, W3, b3, W4, b4, Wl, bl):
    src = edge_index[0]
    dst = edge_index[1]
    edges = jnp.stack([src.reshape(_E // _CH, _CH),
                       dst.reshape(_E // _CH, _CH),
                       lax.bitcast_convert_type(edge_attr, jnp.int32).reshape(_E // _CH, _CH)],
                      axis=1)
    degp = _sc_deg(dst, edge_attr).reshape(_NW, _N)
    dinv2d, sdeg2d = _tc_prep(degp)
    dinv_col = dinv2d.reshape(_N, 1)
    sdeg_col = sdeg2d.reshape(_N, 1)
    y1 = _tc_scale(x, dinv_col)
    cur = [y1]
    pool_tiles = []
    for (W, b) in ((W1, b1), (W2, b2), (W3, b3), (W4, b4)):
        zps = [_sc_agg(t, edges).reshape(_NC, _NPAD, 128)
               for t in cur]
        cur = _tc_layer(zps, cur, dinv_col, W, b.reshape(1, -1))
        pool_tiles.extend(cur)
    pooled = _tc_pool(pool_tiles, sdeg_col, batch.reshape(_N, 1))
    return _tc_final(pooled, Wl, bl.reshape(1, -1))
